# baseline (speedup 1.0000x reference)
import jax
import jax.numpy as jnp
from jax.experimental import pallas as pl

_TOPK = 8
_NE = 64


def _gate_block(x_ref, w_ref, b_ref, idx_ref, wgt_ref):
    x = x_ref[...]
    w = w_ref[...]
    l = jax.lax.dot_general(
        w, x, (((1,), (1,)), ((), ())),
        preferred_element_type=jnp.float32)
    l = l + b_ref[...]

    bm = l.shape[1]
    iota = jax.lax.broadcasted_iota(jnp.int32, (_NE, bm), 0).astype(jnp.float32)
    vals, idxs = [], []
    for _ in range(_TOPK):
        m = jnp.max(l, axis=0, keepdims=True)
        a = jnp.min(jnp.where(l == m, iota, float(_NE)), axis=0,
                    keepdims=True)
        vals.append(m)
        idxs.append(a)
        l = jnp.where(iota == a, -jnp.inf, l)
    v = jnp.concatenate(vals, axis=0)
    i = jnp.concatenate(idxs, axis=0)
    e = jnp.exp(v - v[:1])
    wgt = e / jnp.sum(e, axis=0, keepdims=True)
    idx_ref[...] = i.astype(jnp.int32)
    wgt_ref[...] = wgt


def kernel(hidden_states, weight, e_score_correction_bias):
    x = hidden_states.reshape(-1, hidden_states.shape[-1])
    m, k = x.shape
    bm = 2048
    b2 = e_score_correction_bias.reshape(_NE, 1)
    idx_t, wgt_t = pl.pallas_call(
        _gate_block,
        grid=(m // bm,),
        in_specs=[
            pl.BlockSpec((bm, k), lambda i: (i, 0)),
            pl.BlockSpec((_NE, k), lambda i: (0, 0)),
            pl.BlockSpec((_NE, 1), lambda i: (0, 0)),
        ],
        out_specs=[
            pl.BlockSpec((_TOPK, bm), lambda i: (0, i)),
            pl.BlockSpec((_TOPK, bm), lambda i: (0, i)),
        ],
        out_shape=[
            jax.ShapeDtypeStruct((_TOPK, m), jnp.int32),
            jax.ShapeDtypeStruct((_TOPK, m), jnp.float32),
        ],
    )(x, weight, b2)
    return idx_t.T, wgt_t.T
